# baseline (speedup 1.0000x reference)
import jax
import jax.numpy as jnp
from jax import lax
from jax.experimental import pallas as pl
from jax.experimental.pallas import tpu as pltpu

B, L, D = 16, 4096, 1024


def _body(lens_ref, x_hbm, tok_hbm, ts_hbm,
          out_x_hbm, out_ts_hbm, out_len_ref,
          sem_bulk, sem_small, sem_ts,
          ts_in_v, ts_out_v):
    bulk = [
        pltpu.make_async_copy(x_hbm.at[b], out_x_hbm.at[b, pl.ds(0, L)],
                              sem_bulk.at[b])
        for b in range(B)
    ]
    for cp in bulk:
        cp.start()
    ts_in = pltpu.make_async_copy(ts_hbm, ts_in_v, sem_ts)
    ts_in.start()

    for i in range(B):
        out_len_ref[i] = lens_ref[i] + 1

    ts_in.wait()
    cols = lax.broadcasted_iota(jnp.int32, (1, L), 1)
    for b in range(B):
        last = lens_ref[b]
        last_m1 = jnp.maximum(last - 1, 0)
        row = ts_in_v[b:b + 1, :]
        last_ts = jnp.sum(jnp.where(cols == last_m1, row, 0.0))
        ts_out_v[b:b + 1, :L] = jnp.where(cols == last, last_ts, row)
        ts_out_v[b:b + 1, L:L + 1] = row[:, 0:1]
    ts_out = pltpu.make_async_copy(ts_out_v, out_ts_hbm, sem_ts)
    ts_out.start()

    small = []
    for b in range(B):
        bulk[b].wait()
        last = lens_ref[b]
        cp_tok = pltpu.make_async_copy(
            tok_hbm, out_x_hbm.at[b, pl.ds(last, 1)], sem_small.at[b])
        cp_first = pltpu.make_async_copy(
            x_hbm.at[b, pl.ds(0, 1)], out_x_hbm.at[b, pl.ds(L, 1)],
            sem_small.at[b])
        cp_tok.start()
        cp_first.start()
        small.append((cp_tok, cp_first))

    for cp_tok, cp_first in small:
        cp_tok.wait()
        cp_first.wait()
    ts_out.wait()


def kernel(x_payload, timestamps, seq_lens, token):
    seq_lens = seq_lens.astype(jnp.int32)
    token2 = token.reshape(1, D)

    new_x, new_ts, new_len = pl.pallas_call(
        _body,
        in_specs=[
            pl.BlockSpec(memory_space=pltpu.SMEM),
            pl.BlockSpec(memory_space=pltpu.MemorySpace.HBM),
            pl.BlockSpec(memory_space=pltpu.MemorySpace.HBM),
            pl.BlockSpec(memory_space=pltpu.MemorySpace.HBM),
        ],
        out_specs=[
            pl.BlockSpec(memory_space=pltpu.MemorySpace.HBM),
            pl.BlockSpec(memory_space=pltpu.MemorySpace.HBM),
            pl.BlockSpec(memory_space=pltpu.SMEM),
        ],
        out_shape=[
            jax.ShapeDtypeStruct((B, L + 1, D), x_payload.dtype),
            jax.ShapeDtypeStruct((B, L + 1), timestamps.dtype),
            jax.ShapeDtypeStruct((B,), jnp.int32),
        ],
        scratch_shapes=[
            pltpu.SemaphoreType.DMA((B,)),
            pltpu.SemaphoreType.DMA((B,)),
            pltpu.SemaphoreType.DMA,
            pltpu.VMEM((B, L), jnp.float32),
            pltpu.VMEM((B, L + 1), jnp.float32),
        ],
    )(seq_lens, x_payload, token2, timestamps)
    return new_x, new_len, new_ts, new_len


# speedup vs baseline: 19.3242x; 19.3242x over previous
import jax
import jax.numpy as jnp
from jax import lax
from jax.experimental import pallas as pl
from jax.experimental.pallas import tpu as pltpu

B, L, D = 16, 4096, 1024
CH = 512
NCH = (L + 1 + CH - 1) // CH


def _copy_body(lens_ref, x_ref, first_ref, tok_ref, out_ref):
    b = pl.program_id(0)
    c = pl.program_id(1)
    last = lens_ref[b]
    start = c * CH
    has_tok = (last >= start) & (last < start + CH)
    is_end = c == NCH - 1

    @pl.when(jnp.logical_not(has_tok | is_end))
    def _fast():
        out_ref[...] = x_ref[...]

    @pl.when(has_tok | is_end)
    def _slow():
        rows = lax.broadcasted_iota(jnp.int32, (CH, 1), 0) + start
        y = jnp.where(rows == last, tok_ref[...], x_ref[0])
        y = jnp.where(rows == L, first_ref[0, 0:1], y)
        out_ref[0] = y


def _ts_body(lens_ref, ts_ref, out_ts_ref, out_len_ref):
    cols = lax.broadcasted_iota(jnp.int32, (1, L), 1)
    for b in range(B):
        last = lens_ref[b]
        last_m1 = jnp.maximum(last - 1, 0)
        row = ts_ref[b:b + 1, :]
        last_ts = jnp.sum(jnp.where(cols == last_m1, row, 0.0))
        out_ts_ref[b:b + 1, :L] = jnp.where(cols == last, last_ts, row)
        out_ts_ref[b:b + 1, L:L + 1] = row[:, 0:1]
        out_len_ref[b] = last + 1


def kernel(x_payload, timestamps, seq_lens, token):
    seq_lens = seq_lens.astype(jnp.int32)
    token2 = token.reshape(1, D)

    grid_spec = pltpu.PrefetchScalarGridSpec(
        num_scalar_prefetch=1,
        grid=(B, NCH),
        in_specs=[
            pl.BlockSpec((1, CH, D),
                         lambda b, c, lens: (b, jnp.minimum(c, NCH - 2), 0)),
            pl.BlockSpec((1, 8, D), lambda b, c, lens: (b, 0, 0)),
            pl.BlockSpec((1, D), lambda b, c, lens: (0, 0)),
        ],
        out_specs=pl.BlockSpec((1, CH, D), lambda b, c, lens: (b, c, 0)),
    )
    new_x = pl.pallas_call(
        _copy_body,
        grid_spec=grid_spec,
        out_shape=jax.ShapeDtypeStruct((B, L + 1, D), x_payload.dtype),
        compiler_params=pltpu.CompilerParams(
            dimension_semantics=("parallel", "parallel"),
        ),
    )(seq_lens, x_payload, x_payload, token2)

    new_ts, new_len = pl.pallas_call(
        _ts_body,
        in_specs=[
            pl.BlockSpec(memory_space=pltpu.SMEM),
            pl.BlockSpec(memory_space=pltpu.VMEM),
        ],
        out_specs=[
            pl.BlockSpec(memory_space=pltpu.VMEM),
            pl.BlockSpec(memory_space=pltpu.SMEM),
        ],
        out_shape=[
            jax.ShapeDtypeStruct((B, L + 1), timestamps.dtype),
            jax.ShapeDtypeStruct((B,), jnp.int32),
        ],
    )(seq_lens, timestamps)
    return new_x, new_len, new_ts, new_len


# speedup vs baseline: 20.1639x; 1.0435x over previous
import jax
import jax.numpy as jnp
from jax import lax
from jax.experimental import pallas as pl
from jax.experimental.pallas import tpu as pltpu

B, L, D = 16, 4096, 1024
CH = 1024
NCH = (L + 1 + CH - 1) // CH


def _copy_body(lens_ref, x_ref, first_ref, tok_ref, out_ref):
    b = pl.program_id(0)
    c = pl.program_id(1)
    last = lens_ref[b]
    start = c * CH
    has_tok = (last >= start) & (last < start + CH)
    is_end = c == NCH - 1

    @pl.when(jnp.logical_not(has_tok | is_end))
    def _fast():
        out_ref[...] = x_ref[...]

    @pl.when(has_tok | is_end)
    def _slow():
        rows = lax.broadcasted_iota(jnp.int32, (CH, 1), 0) + start
        y = jnp.where(rows == last, tok_ref[...], x_ref[0])
        y = jnp.where(rows == L, first_ref[0, 0:1], y)
        out_ref[0] = y


def _ts_body(lens_ref, ts_ref, out_ts_ref, out_len_ref):
    cols = lax.broadcasted_iota(jnp.int32, (1, L), 1)
    for b in range(B):
        last = lens_ref[b]
        last_m1 = jnp.maximum(last - 1, 0)
        row = ts_ref[b:b + 1, :]
        last_ts = jnp.sum(jnp.where(cols == last_m1, row, 0.0))
        out_ts_ref[b:b + 1, :L] = jnp.where(cols == last, last_ts, row)
        out_ts_ref[b:b + 1, L:L + 1] = row[:, 0:1]
        out_len_ref[b] = last + 1


def kernel(x_payload, timestamps, seq_lens, token):
    seq_lens = seq_lens.astype(jnp.int32)
    token2 = token.reshape(1, D)

    grid_spec = pltpu.PrefetchScalarGridSpec(
        num_scalar_prefetch=1,
        grid=(B, NCH),
        in_specs=[
            pl.BlockSpec((1, CH, D),
                         lambda b, c, lens: (b, jnp.minimum(c, NCH - 2), 0)),
            pl.BlockSpec((1, 8, D), lambda b, c, lens: (b, 0, 0)),
            pl.BlockSpec((1, D), lambda b, c, lens: (0, 0)),
        ],
        out_specs=pl.BlockSpec((1, CH, D), lambda b, c, lens: (b, c, 0)),
    )
    new_x = pl.pallas_call(
        _copy_body,
        grid_spec=grid_spec,
        out_shape=jax.ShapeDtypeStruct((B, L + 1, D), x_payload.dtype),
        compiler_params=pltpu.CompilerParams(
            dimension_semantics=("parallel", "parallel"),
        ),
    )(seq_lens, x_payload, x_payload, token2)

    new_ts, new_len = pl.pallas_call(
        _ts_body,
        in_specs=[
            pl.BlockSpec(memory_space=pltpu.SMEM),
            pl.BlockSpec(memory_space=pltpu.VMEM),
        ],
        out_specs=[
            pl.BlockSpec(memory_space=pltpu.VMEM),
            pl.BlockSpec(memory_space=pltpu.SMEM),
        ],
        out_shape=[
            jax.ShapeDtypeStruct((B, L + 1), timestamps.dtype),
            jax.ShapeDtypeStruct((B,), jnp.int32),
        ],
    )(seq_lens, timestamps)
    return new_x, new_len, new_ts, new_len


# speedup vs baseline: 20.9529x; 1.0391x over previous
import jax
import jax.numpy as jnp
from jax import lax
from jax.experimental import pallas as pl
from jax.experimental.pallas import tpu as pltpu

B, L, D = 16, 4096, 1024
CH = 2048
NCH = (L + 1 + CH - 1) // CH


def _copy_body(lens_ref, x_ref, first_ref, tok_ref, out_ref):
    b = pl.program_id(0)
    c = pl.program_id(1)
    last = lens_ref[b]
    start = c * CH
    has_tok = (last >= start) & (last < start + CH)
    is_end = c == NCH - 1

    @pl.when(jnp.logical_not(has_tok | is_end))
    def _fast():
        out_ref[...] = x_ref[...]

    @pl.when(has_tok | is_end)
    def _slow():
        rows = lax.broadcasted_iota(jnp.int32, (CH, 1), 0) + start
        y = jnp.where(rows == last, tok_ref[...], x_ref[0])
        y = jnp.where(rows == L, first_ref[0, 0:1], y)
        out_ref[0] = y


def _ts_body(lens_ref, ts_ref, out_ts_ref, out_len_ref):
    cols = lax.broadcasted_iota(jnp.int32, (1, L), 1)
    for b in range(B):
        last = lens_ref[b]
        last_m1 = jnp.maximum(last - 1, 0)
        row = ts_ref[b:b + 1, :]
        last_ts = jnp.sum(jnp.where(cols == last_m1, row, 0.0))
        out_ts_ref[b:b + 1, :L] = jnp.where(cols == last, last_ts, row)
        out_ts_ref[b:b + 1, L:L + 1] = row[:, 0:1]
        out_len_ref[b] = last + 1


def kernel(x_payload, timestamps, seq_lens, token):
    seq_lens = seq_lens.astype(jnp.int32)
    token2 = token.reshape(1, D)

    grid_spec = pltpu.PrefetchScalarGridSpec(
        num_scalar_prefetch=1,
        grid=(B, NCH),
        in_specs=[
            pl.BlockSpec((1, CH, D),
                         lambda b, c, lens: (b, jnp.minimum(c, NCH - 2), 0)),
            pl.BlockSpec((1, 8, D), lambda b, c, lens: (b, 0, 0)),
            pl.BlockSpec((1, D), lambda b, c, lens: (0, 0)),
        ],
        out_specs=pl.BlockSpec((1, CH, D), lambda b, c, lens: (b, c, 0)),
    )
    new_x = pl.pallas_call(
        _copy_body,
        grid_spec=grid_spec,
        out_shape=jax.ShapeDtypeStruct((B, L + 1, D), x_payload.dtype),
        compiler_params=pltpu.CompilerParams(
            dimension_semantics=("parallel", "parallel"),
        ),
    )(seq_lens, x_payload, x_payload, token2)

    new_ts, new_len = pl.pallas_call(
        _ts_body,
        in_specs=[
            pl.BlockSpec(memory_space=pltpu.SMEM),
            pl.BlockSpec(memory_space=pltpu.VMEM),
        ],
        out_specs=[
            pl.BlockSpec(memory_space=pltpu.VMEM),
            pl.BlockSpec(memory_space=pltpu.SMEM),
        ],
        out_shape=[
            jax.ShapeDtypeStruct((B, L + 1), timestamps.dtype),
            jax.ShapeDtypeStruct((B,), jnp.int32),
        ],
    )(seq_lens, timestamps)
    return new_x, new_len, new_ts, new_len


# speedup vs baseline: 21.1990x; 1.0117x over previous
import jax
import jax.numpy as jnp
from jax import lax
from jax.experimental import pallas as pl
from jax.experimental.pallas import tpu as pltpu

B, L, D = 16, 4096, 1024
DC = 512


def _copy_body(lens_ref, x_ref, tok_ref, out_ref):
    b = pl.program_id(0)
    last = lens_ref[b]
    out_ref[0, :L] = x_ref[0]
    out_ref[0, L:L + 1] = x_ref[0, 0:1]
    out_ref[0, pl.ds(last, 1)] = tok_ref[...]


def _ts_body(lens_ref, ts_ref, out_ts_ref, out_len_ref):
    cols = lax.broadcasted_iota(jnp.int32, (1, L), 1)
    for b in range(B):
        last = lens_ref[b]
        last_m1 = jnp.maximum(last - 1, 0)
        row = ts_ref[b:b + 1, :]
        last_ts = jnp.sum(jnp.where(cols == last_m1, row, 0.0))
        out_ts_ref[b:b + 1, :L] = jnp.where(cols == last, last_ts, row)
        out_ts_ref[b:b + 1, L:L + 1] = row[:, 0:1]
        out_len_ref[b] = last + 1


def kernel(x_payload, timestamps, seq_lens, token):
    seq_lens = seq_lens.astype(jnp.int32)
    token2 = token.reshape(1, D)

    grid_spec = pltpu.PrefetchScalarGridSpec(
        num_scalar_prefetch=1,
        grid=(B, D // DC),
        in_specs=[
            pl.BlockSpec((1, L, DC), lambda b, d, lens: (b, 0, d)),
            pl.BlockSpec((1, DC), lambda b, d, lens: (0, d)),
        ],
        out_specs=pl.BlockSpec((1, L + 1, DC), lambda b, d, lens: (b, 0, d)),
    )
    new_x = pl.pallas_call(
        _copy_body,
        grid_spec=grid_spec,
        out_shape=jax.ShapeDtypeStruct((B, L + 1, D), x_payload.dtype),
        compiler_params=pltpu.CompilerParams(
            dimension_semantics=("parallel", "parallel"),
        ),
    )(seq_lens, x_payload, token2)

    new_ts, new_len = pl.pallas_call(
        _ts_body,
        in_specs=[
            pl.BlockSpec(memory_space=pltpu.SMEM),
            pl.BlockSpec(memory_space=pltpu.VMEM),
        ],
        out_specs=[
            pl.BlockSpec(memory_space=pltpu.VMEM),
            pl.BlockSpec(memory_space=pltpu.SMEM),
        ],
        out_shape=[
            jax.ShapeDtypeStruct((B, L + 1), timestamps.dtype),
            jax.ShapeDtypeStruct((B,), jnp.int32),
        ],
    )(seq_lens, timestamps)
    return new_x, new_len, new_ts, new_len
